# Initial kernel scaffold; baseline (speedup 1.0000x reference)
import functools

import jax
import jax.numpy as jnp
from jax import lax
from jax.experimental import pallas as pl
from jax.experimental.pallas import tpu as pltpu
from jax.experimental.pallas import tpu_sc as plsc

N = 10000
NB = 2
NN = N * NB
NR = 20016
DUMMY = 20000
E = 160000
SUB = 128
NTILES = 32
E1P = 163840
E2P = 327680
J1 = E1P // (NTILES * SUB)
J2 = E2P // (NTILES * SUB)
ND = 10016
NDT = ND // 16
NRT = NR // 16
F = 16
IN_C = 2
OUT_C = 16

_MESH = dict(mesh=plsc.VectorSubcoreMesh(core_axis_name="c", subcore_axis_name="s"))


def _fill_zeros(buf, n):
    z = jnp.zeros((F,), jnp.float32)

    def body(i, _):
        buf[i, :] = z
        return 0

    lax.fori_loop(0, n, body, 0)


def _sc_preamble(rowp, colp, wp):
    def body(row_h, col_h, w_h, dinv_o, wr_o,
             dego_s, degi_s, rbuf, rows, rowb, colb, wb, r4):
        c = lax.axis_index("c")
        s = lax.axis_index("s")
        wid = c * 16 + s
        _fill_zeros(rbuf, NDT)
        pltpu.sync_copy(rbuf, dego_s.at[pl.ds(s * NDT, NDT)])
        pltpu.sync_copy(rbuf, degi_s.at[pl.ds(s * NDT, NDT)])
        pltpu.sync_copy(row_h.at[pl.ds(s * (2 * J1), 2 * J1)], rowb)
        pltpu.sync_copy(col_h.at[pl.ds(s * (2 * J1), 2 * J1)], colb)
        pltpu.sync_copy(w_h.at[pl.ds(s * (2 * J1), 2 * J1)], wb)
        plsc.subcore_barrier()

        def deg_body(j, _):
            for i in range(SUB):
                rows[i, :] = jnp.full((F,), wb[j, i], jnp.float32)
            pltpu.sync_copy(rows, dego_s.at[rowb.at[j]], add=True)
            pltpu.sync_copy(rows, degi_s.at[colb.at[j]], add=True)
            return 0

        lax.fori_loop(0, 2 * J1, deg_body, 0)
        plsc.subcore_barrier()

        def recip(i, _):
            rbuf[i, :] = 1.0 / rbuf[i, :]
            return 0

        pltpu.sync_copy(dego_s.at[pl.ds(s * NDT, NDT)], rbuf)
        lax.fori_loop(0, NDT, recip, 0)

        @pl.when(c == 0)
        def _():
            pltpu.sync_copy(rbuf, dinv_o.at[0, pl.ds(s * NDT, NDT)])

        pltpu.sync_copy(degi_s.at[pl.ds(s * NDT, NDT)], rbuf)
        lax.fori_loop(0, NDT, recip, 0)
        pltpu.sync_copy(rbuf, degi_s.at[pl.ds(s * NDT, NDT)])

        @pl.when(c == 0)
        def _():
            pltpu.sync_copy(rbuf, dinv_o.at[1, pl.ds(s * NDT, NDT)])

        plsc.subcore_barrier()

        pltpu.sync_copy(row_h.at[pl.ds(wid * J1, J1)], r4)

        def wr_body(j, _):
            pltpu.sync_copy(degi_s.at[r4.at[j]], rows)
            k = wid * (J1 * SUB) + j * SUB
            pltpu.sync_copy(rows, wr_o.at[pl.ds(k, SUB)])
            pltpu.sync_copy(rows, wr_o.at[pl.ds(E1P + k, SUB)])
            return 0

        lax.fori_loop(0, J1, wr_body, 0)

    fn = pl.kernel(
        body,
        out_type=(
            jax.ShapeDtypeStruct((2, ND, F), jnp.float32),
            jax.ShapeDtypeStruct((E2P, F), jnp.float32),
        ),
        scratch_types=(
            pltpu.VMEM_SHARED((ND, F), jnp.float32),
            pltpu.VMEM_SHARED((ND, F), jnp.float32),
            pltpu.VMEM((NDT, F), jnp.float32),
            pltpu.VMEM((SUB, F), jnp.float32),
            pltpu.VMEM((2 * J1, SUB), jnp.int32),
            pltpu.VMEM((2 * J1, SUB), jnp.int32),
            pltpu.VMEM((2 * J1, SUB), jnp.float32),
            pltpu.VMEM((J1, SUB), jnp.int32),
        ),
        **_MESH,
    )
    return fn(rowp, colp, wp)


def _sc_prop(xs, xr, srcf, dstf, srcr, dstr, wr):
    def body(xs_h, xr_h, sf_h, df_h, sr_h, dr_h, wr_h, yo_o, yi_o,
             acco, acci, zbuf, rows, wbuf, sfb, dfb, srb, drb):
        c = lax.axis_index("c")
        s = lax.axis_index("s")
        wid = c * 16 + s
        _fill_zeros(zbuf, NRT)
        pltpu.sync_copy(zbuf, acco.at[pl.ds(s * NRT, NRT)])
        pltpu.sync_copy(zbuf, acci.at[pl.ds(s * NRT, NRT)])
        pltpu.sync_copy(sf_h.at[pl.ds(wid * J2, J2)], sfb)
        pltpu.sync_copy(df_h.at[pl.ds(wid * J2, J2)], dfb)
        pltpu.sync_copy(sr_h.at[pl.ds(wid * J2, J2)], srb)
        pltpu.sync_copy(dr_h.at[pl.ds(wid * J2, J2)], drb)
        plsc.subcore_barrier()

        def fwd(j, _):
            pltpu.sync_copy(xs_h.at[sfb.at[j]], rows)
            pltpu.sync_copy(rows, acco.at[dfb.at[j]], add=True)
            return 0

        lax.fori_loop(0, J2, fwd, 0)

        def rev(j, _):
            pltpu.sync_copy(xr_h.at[srb.at[j]], rows)
            pltpu.sync_copy(wr_h.at[pl.ds(wid * (J2 * SUB) + j * SUB, SUB)],
                            wbuf)
            for i in range(SUB):
                rows[i, :] = rows[i, :] * wbuf[i, :]
            pltpu.sync_copy(rows, acci.at[drb.at[j]], add=True)
            return 0

        lax.fori_loop(0, J2, rev, 0)
        plsc.subcore_barrier()
        pltpu.sync_copy(acco.at[pl.ds(s * NRT, NRT)],
                        yo_o.at[c, pl.ds(s * NRT, NRT)])
        pltpu.sync_copy(acci.at[pl.ds(s * NRT, NRT)],
                        yi_o.at[c, pl.ds(s * NRT, NRT)])

    fn = pl.kernel(
        body,
        out_type=(
            jax.ShapeDtypeStruct((2, NR, F), jnp.float32),
            jax.ShapeDtypeStruct((2, NR, F), jnp.float32),
        ),
        scratch_types=(
            pltpu.VMEM_SHARED((NR, F), jnp.float32),
            pltpu.VMEM_SHARED((NR, F), jnp.float32),
            pltpu.VMEM((NRT, F), jnp.float32),
            pltpu.VMEM((SUB, F), jnp.float32),
            pltpu.VMEM((SUB, F), jnp.float32),
            pltpu.VMEM((J2, SUB), jnp.int32),
            pltpu.VMEM((J2, SUB), jnp.int32),
            pltpu.VMEM((J2, SUB), jnp.int32),
            pltpu.VMEM((J2, SUB), jnp.int32),
        ),
        **_MESH,
    )
    return fn(xs, xr, srcf, dstf, srcr, dstr, wr)


def _small_mm(x, w):
    acc = x[:, 0:1] * w[0][None, :]
    for i in range(1, w.shape[0]):
        acc = acc + x[:, i:i + 1] * w[i][None, :]
    return acc


def _tc_scale_kernel(xt_ref, dinv_ref, dinvb_ref, xs_ref):
    do = dinv_ref[0, :N, :]
    dinvb = jnp.concatenate([do, do, jnp.zeros((NR - NN, F), jnp.float32)],
                            axis=0)
    dinvb_ref[...] = dinvb
    xs_ref[...] = xt_ref[...] * dinvb


def _tc_scale(x_tab, dinv):
    return pl.pallas_call(
        _tc_scale_kernel,
        out_shape=(jax.ShapeDtypeStruct((NR, F), jnp.float32),
                   jax.ShapeDtypeStruct((NR, F), jnp.float32)),
    )(x_tab, dinv)


def _gate_pre(xcols, hx, poxt, pixt, yo, yi, wpk):
    a2, a16, bo2, bo16, ci2, ci16, b = wpk
    return (_small_mm(xcols, a2) + jnp.dot(hx, a16,
                                           preferred_element_type=jnp.float32)
            + _small_mm(poxt, bo2)
            + jnp.dot(yo, bo16, preferred_element_type=jnp.float32)
            + _small_mm(pixt, ci2)
            + jnp.dot(yi, ci16, preferred_element_type=jnp.float32)
            + b[None, :])


def _tc_gates_kernel(t, xt_ref, pox_ref, pix_ref, h_ref, yo_ref, yi_ref,
                     dinvb_ref, wz_refs, wr_refs, z_ref, hr_ref, hrs_ref):
    xcols = xt_ref[:, 2 * t:2 * t + 2]
    poxt = (pox_ref[0] + pox_ref[1])[:, 2 * t:2 * t + 2]
    pixt = (pix_ref[0] + pix_ref[1])[:, 2 * t:2 * t + 2]
    h = h_ref[...]
    yo = yo_ref[0] + yo_ref[1]
    yi = yi_ref[0] + yi_ref[1]
    wz = [r[...] for r in wz_refs]
    wr = [r[...] for r in wr_refs]
    z = jax.nn.sigmoid(_gate_pre(xcols, h, poxt, pixt, yo, yi, wz))
    r = jax.nn.sigmoid(_gate_pre(xcols, h, poxt, pixt, yo, yi, wr))
    hr = h * r
    z_ref[...] = z
    hr_ref[...] = hr
    hrs_ref[...] = hr * dinvb_ref[...]


def _tc_gates(t, x_tab, pox, pix, h, yo, yi, dinvb, wz_pack, wr_pack):
    return pl.pallas_call(
        functools.partial(_tc_gates_kernel, t),
        out_shape=(jax.ShapeDtypeStruct((NR, F), jnp.float32),
                   jax.ShapeDtypeStruct((NR, F), jnp.float32),
                   jax.ShapeDtypeStruct((NR, F), jnp.float32)),
    )(x_tab, pox, pix, h, yo, yi, dinvb, wz_pack, wr_pack)


def _tc_update_kernel(t, xt_ref, pox_ref, pix_ref, h_ref, z_ref, hr_ref,
                      yo_ref, yi_ref, dinvb_ref, wh_refs,
                      hn_ref, hns_ref):
    xcols = xt_ref[:, 2 * t:2 * t + 2]
    poxt = (pox_ref[0] + pox_ref[1])[:, 2 * t:2 * t + 2]
    pixt = (pix_ref[0] + pix_ref[1])[:, 2 * t:2 * t + 2]
    yo = yo_ref[0] + yo_ref[1]
    yi = yi_ref[0] + yi_ref[1]
    wh = [r[...] for r in wh_refs]
    pre = _gate_pre(xcols, hr_ref[...], poxt, pixt, yo, yi, wh)
    z = z_ref[...]
    hn = z * h_ref[...] + (1.0 - z) * jnp.tanh(pre)
    hn_ref[...] = hn
    hns_ref[...] = hn * dinvb_ref[...]


def _tc_update(t, x_tab, pox, pix, h, z, hr, yo, yi, dinvb, wh_pack):
    return pl.pallas_call(
        functools.partial(_tc_update_kernel, t),
        out_shape=(jax.ShapeDtypeStruct((NR, F), jnp.float32),
                   jax.ShapeDtypeStruct((NR, F), jnp.float32)),
    )(x_tab, pox, pix, h, z, hr, yo, yi, dinvb, wh_pack)


def _pack_w(w, b):
    a = w[0, 0] + w[1, 0]
    return (a[:IN_C], a[IN_C:], w[0, 1][:IN_C], w[0, 1][IN_C:],
            w[1, 1][:IN_C], w[1, 1][IN_C:], b)


def kernel(X, edge_index, edge_weight, Wz, bz, Wr, br, Wh, bh):
    Bs, Ts, Ns, Fin = X.shape
    row = edge_index[0].astype(jnp.int32)
    col = edge_index[1].astype(jnp.int32)
    w = edge_weight

    perm = jnp.argsort(col * Ns + row)
    src_r = col[perm]
    dst_r = row[perm]
    pad1 = E1P - E
    padi = jnp.full((pad1,), N, jnp.int32)
    padd = jnp.full((pad1,), DUMMY, jnp.int32)

    rowp = jnp.concatenate([row, padi]).reshape(E1P // SUB, SUB)
    colp = jnp.concatenate([col, padi]).reshape(E1P // SUB, SUB)
    wp = jnp.concatenate([w, jnp.zeros((pad1,), jnp.float32)]
                         ).reshape(E1P // SUB, SUB)

    def batchpad(a):
        return jnp.concatenate(
            [a, padd, a + Ns, padd]).reshape(E2P // SUB, SUB)

    srcf, dstf = batchpad(row), batchpad(col)
    srcr, dstr = batchpad(src_r), batchpad(dst_r)

    x_tab = X.transpose(0, 2, 1, 3).reshape(NN, Ts * Fin)
    x_tab = jnp.pad(x_tab, ((0, NR - NN), (0, F - Ts * Fin)))

    wz_pack = _pack_w(Wz, bz)
    wr_pack = _pack_w(Wr, br)
    wh_pack = _pack_w(Wh, bh)

    dinv, wr_rows = _sc_preamble(rowp, colp, wp)

    dinvb, xs_x = _tc_scale(x_tab, dinv)
    pox, pix = _sc_prop(xs_x, x_tab, srcf, dstf, srcr, dstr, wr_rows)

    h = jnp.zeros((NR, F), jnp.float32)
    h_s = h
    zero_part = jnp.zeros((2, NR, F), jnp.float32)
    outs = []
    for t in range(Ts):
        if t == 0:
            yo_h = yi_h = zero_part
        else:
            yo_h, yi_h = _sc_prop(h_s, h, srcf, dstf, srcr, dstr, wr_rows)
        z, hr, hr_s = _tc_gates(t, x_tab, pox, pix, h, yo_h, yi_h, dinvb,
                                wz_pack, wr_pack)
        yo_c, yi_c = _sc_prop(hr_s, hr, srcf, dstf, srcr, dstr, wr_rows)
        h, h_s = _tc_update(t, x_tab, pox, pix, h, z, hr, yo_c, yi_c, dinvb,
                            wh_pack)
        outs.append(h[:NN].reshape(Bs, Ns, OUT_C))
    return jnp.stack(outs, axis=1)


# speedup vs baseline: 8.4819x; 8.4819x over previous
import functools

import jax
import jax.numpy as jnp
from jax import lax
from jax.experimental import pallas as pl
from jax.experimental.pallas import tpu as pltpu
from jax.experimental.pallas import tpu_sc as plsc

N = 10000
NB = 2
NN = N * NB
NR = 20096
DUMMY = 20000
E = 160000
SUB = 128
NTILES = 32
E1P = 163840
E2P = 327680
J1 = E1P // (NTILES * SUB)
J2 = E2P // (NTILES * SUB)
ND = 10112
NDT = ND // 16
NRT = NR // 16
F = 16
IN_C = 2
OUT_C = 16
JW = 16

_MESH = dict(
    mesh=plsc.VectorSubcoreMesh(core_axis_name="c", subcore_axis_name="s"),
    compiler_params=pltpu.CompilerParams(use_tc_tiling_on_sc=False),
)


def _sc_preamble(rowp, colp, wp, zeros_h):
    def body(row_h, col_h, w_h, z_h, dinv_o, wr_o,
             dego_s, degi_s, rbuf, rows, rowb, colb, wb, r4):
        c = lax.axis_index("c")
        s = lax.axis_index("s")
        wid = c * 16 + s
        pltpu.sync_copy(z_h.at[pl.ds(s * NDT, NDT)],
                        dego_s.at[pl.ds(s * NDT, NDT)])
        pltpu.sync_copy(z_h.at[pl.ds(s * NDT, NDT)],
                        degi_s.at[pl.ds(s * NDT, NDT)])
        plsc.subcore_barrier()

        def deg_body(j, _):
            for g in range(SUB // F):
                wv = wb[j, pl.ds(g * F, F)]
                for l in range(F):
                    rows[g * F + l, :] = jnp.full((F,), wv[l], jnp.float32)
            pltpu.sync_copy(rows, dego_s.at[rowb.at[j]], add=True)
            pltpu.sync_copy(rows, degi_s.at[colb.at[j]], add=True)
            return 0

        for wj in range(2 * J1 // JW):
            pltpu.sync_copy(row_h.at[pl.ds(s * (2 * J1) + wj * JW, JW)], rowb)
            pltpu.sync_copy(col_h.at[pl.ds(s * (2 * J1) + wj * JW, JW)], colb)
            pltpu.sync_copy(w_h.at[pl.ds(s * (2 * J1) + wj * JW, JW)], wb)
            lax.fori_loop(0, JW, deg_body, 0)
        plsc.subcore_barrier()

        def recip(i, _):
            rbuf[i, :] = 1.0 / rbuf[i, :]
            return 0

        pltpu.sync_copy(dego_s.at[pl.ds(s * NDT, NDT)], rbuf)
        lax.fori_loop(0, NDT, recip, 0)

        @pl.when(c == 0)
        def _():
            pltpu.sync_copy(rbuf, dinv_o.at[0, pl.ds(s * NDT, NDT)])

        pltpu.sync_copy(degi_s.at[pl.ds(s * NDT, NDT)], rbuf)
        lax.fori_loop(0, NDT, recip, 0)
        pltpu.sync_copy(rbuf, degi_s.at[pl.ds(s * NDT, NDT)])

        @pl.when(c == 0)
        def _():
            pltpu.sync_copy(rbuf, dinv_o.at[1, pl.ds(s * NDT, NDT)])

        plsc.subcore_barrier()

        pltpu.sync_copy(row_h.at[pl.ds(wid * J1, J1)], r4)

        def wr_body(j, _):
            pltpu.sync_copy(degi_s.at[r4.at[j]], rows)
            k = wid * (J1 * SUB) + j * SUB
            pltpu.sync_copy(rows, wr_o.at[pl.ds(k, SUB)])
            pltpu.sync_copy(rows, wr_o.at[pl.ds(E1P + k, SUB)])
            return 0

        lax.fori_loop(0, J1, wr_body, 0)

    fn = pl.kernel(
        body,
        out_type=(
            jax.ShapeDtypeStruct((2, ND, F), jnp.float32),
            jax.ShapeDtypeStruct((E2P, F), jnp.float32),
        ),
        scratch_types=(
            pltpu.VMEM_SHARED((ND, F), jnp.float32),
            pltpu.VMEM_SHARED((ND, F), jnp.float32),
            pltpu.VMEM((NDT, F), jnp.float32),
            pltpu.VMEM((SUB, F), jnp.float32),
            pltpu.VMEM((JW, SUB), jnp.int32),
            pltpu.VMEM((JW, SUB), jnp.int32),
            pltpu.VMEM((JW, SUB), jnp.float32),
            pltpu.VMEM((J1, SUB), jnp.int32),
        ),
        **_MESH,
    )
    return fn(rowp, colp, wp, zeros_h)


def _sc_prop(xs, xr, srcf, dstf, srcr, dstr, wr, zeros_h):
    def body(xs_h, xr_h, sf_h, df_h, sr_h, dr_h, wr_h, z_h, yo_o, yi_o,
             acco, acci, rows, wbuf, sb, db):
        c = lax.axis_index("c")
        s = lax.axis_index("s")
        wid = c * 16 + s
        pltpu.sync_copy(z_h.at[pl.ds(s * NRT, NRT)],
                        acco.at[pl.ds(s * NRT, NRT)])
        pltpu.sync_copy(z_h.at[pl.ds(s * NRT, NRT)],
                        acci.at[pl.ds(s * NRT, NRT)])
        plsc.subcore_barrier()

        def fwd(j, _):
            pltpu.sync_copy(xs_h.at[sb.at[j]], rows)
            pltpu.sync_copy(rows, acco.at[db.at[j]], add=True)
            return 0

        def rev(wj):
            def inner(j, _):
                pltpu.sync_copy(xr_h.at[sb.at[j]], rows)
                pltpu.sync_copy(
                    wr_h.at[pl.ds(wid * (J2 * SUB) + (wj * JW + j) * SUB,
                                  SUB)], wbuf)
                for i in range(SUB):
                    rows[i, :] = rows[i, :] * wbuf[i, :]
                pltpu.sync_copy(rows, acci.at[db.at[j]], add=True)
                return 0
            return inner

        for wj in range(J2 // JW):
            pltpu.sync_copy(sf_h.at[pl.ds(wid * J2 + wj * JW, JW)], sb)
            pltpu.sync_copy(df_h.at[pl.ds(wid * J2 + wj * JW, JW)], db)
            lax.fori_loop(0, JW, fwd, 0)
        for wj in range(J2 // JW):
            pltpu.sync_copy(sr_h.at[pl.ds(wid * J2 + wj * JW, JW)], sb)
            pltpu.sync_copy(dr_h.at[pl.ds(wid * J2 + wj * JW, JW)], db)
            lax.fori_loop(0, JW, rev(wj), 0)
        plsc.subcore_barrier()
        pltpu.sync_copy(acco.at[pl.ds(s * NRT, NRT)],
                        yo_o.at[c, pl.ds(s * NRT, NRT)])
        pltpu.sync_copy(acci.at[pl.ds(s * NRT, NRT)],
                        yi_o.at[c, pl.ds(s * NRT, NRT)])

    fn = pl.kernel(
        body,
        out_type=(
            jax.ShapeDtypeStruct((2, NR, F), jnp.float32),
            jax.ShapeDtypeStruct((2, NR, F), jnp.float32),
        ),
        scratch_types=(
            pltpu.VMEM_SHARED((NR, F), jnp.float32),
            pltpu.VMEM_SHARED((NR, F), jnp.float32),
            pltpu.VMEM((SUB, F), jnp.float32),
            pltpu.VMEM((SUB, F), jnp.float32),
            pltpu.VMEM((JW, SUB), jnp.int32),
            pltpu.VMEM((JW, SUB), jnp.int32),
        ),
        **_MESH,
    )
    return fn(xs, xr, srcf, dstf, srcr, dstr, wr, zeros_h)


NR8 = NR // 8
ND8 = ND // 8
N8 = N // 8


def _tc_scale_kernel(xt_ref, dinv_ref, dinvb_ref, xs_ref):
    do = dinv_ref[0, :N8, :]
    dinvb = jnp.concatenate(
        [do, do, jnp.zeros((NR8 - 2 * N8, 128), jnp.float32)], axis=0)
    dinvb_ref[...] = dinvb
    xs_ref[...] = xt_ref[...] * dinvb


def _tc_scale(x_tab8, dinv8):
    return pl.pallas_call(
        _tc_scale_kernel,
        out_shape=(jax.ShapeDtypeStruct((NR8, 128), jnp.float32),
                   jax.ShapeDtypeStruct((NR8, 128), jnp.float32)),
    )(x_tab8, dinv8)


def _gate_pre(x8, h8, pox8, pix8, yo8, yi8, w):
    wxa, wha, wxb, wob, wxc, wic, b = w
    f32 = jnp.float32
    return (jnp.dot(x8, wxa, preferred_element_type=f32)
            + jnp.dot(h8, wha, preferred_element_type=f32)
            + jnp.dot(pox8, wxb, preferred_element_type=f32)
            + jnp.dot(yo8, wob, preferred_element_type=f32)
            + jnp.dot(pix8, wxc, preferred_element_type=f32)
            + jnp.dot(yi8, wic, preferred_element_type=f32)
            + b[None, :])


def _tc_gates_kernel(xt_ref, pox_ref, pix_ref, h_ref, yo_ref, yi_ref,
                     dinvb_ref, wz_refs, wr_refs, z_ref, hr_ref, hrs_ref):
    x8 = xt_ref[...]
    pox8 = pox_ref[0] + pox_ref[1]
    pix8 = pix_ref[0] + pix_ref[1]
    h = h_ref[...]
    yo = yo_ref[0] + yo_ref[1]
    yi = yi_ref[0] + yi_ref[1]
    wz = [r[...] for r in wz_refs]
    wr = [r[...] for r in wr_refs]
    z = jax.nn.sigmoid(_gate_pre(x8, h, pox8, pix8, yo, yi, wz))
    r = jax.nn.sigmoid(_gate_pre(x8, h, pox8, pix8, yo, yi, wr))
    hr = h * r
    z_ref[...] = z
    hr_ref[...] = hr
    hrs_ref[...] = hr * dinvb_ref[...]


def _tc_gates(x_tab8, pox, pix, h, yo, yi, dinvb, wz_pack, wr_pack):
    return pl.pallas_call(
        _tc_gates_kernel,
        out_shape=(jax.ShapeDtypeStruct((NR8, 128), jnp.float32),
                   jax.ShapeDtypeStruct((NR8, 128), jnp.float32),
                   jax.ShapeDtypeStruct((NR8, 128), jnp.float32)),
    )(x_tab8, pox, pix, h, yo, yi, dinvb, wz_pack, wr_pack)


def _tc_update_kernel(xt_ref, pox_ref, pix_ref, h_ref, z_ref, hr_ref,
                      yo_ref, yi_ref, dinvb_ref, wh_refs, hn_ref, hns_ref):
    x8 = xt_ref[...]
    pox8 = pox_ref[0] + pox_ref[1]
    pix8 = pix_ref[0] + pix_ref[1]
    yo = yo_ref[0] + yo_ref[1]
    yi = yi_ref[0] + yi_ref[1]
    wh = [r[...] for r in wh_refs]
    pre = _gate_pre(x8, hr_ref[...], pox8, pix8, yo, yi, wh)
    z = z_ref[...]
    hn = z * h_ref[...] + (1.0 - z) * jnp.tanh(pre)
    hn_ref[...] = hn
    hns_ref[...] = hn * dinvb_ref[...]


def _tc_update(x_tab8, pox, pix, h, z, hr, yo, yi, dinvb, wh_pack):
    return pl.pallas_call(
        _tc_update_kernel,
        out_shape=(jax.ShapeDtypeStruct((NR8, 128), jnp.float32),
                   jax.ShapeDtypeStruct((NR8, 128), jnp.float32)),
    )(x_tab8, pox, pix, h, z, hr, yo, yi, dinvb, wh_pack)


def _bd(m):
    return jnp.kron(jnp.eye(8, dtype=jnp.float32), m)


def _pack_w(w, b, t):
    a = w[0, 0] + w[1, 0]

    def pt(m2):
        return jnp.zeros((F, F), jnp.float32).at[2 * t:2 * t + 2].set(m2)

    return (_bd(pt(a[:IN_C])), _bd(a[IN_C:]),
            _bd(pt(w[0, 1][:IN_C])), _bd(w[0, 1][IN_C:]),
            _bd(pt(w[1, 1][:IN_C])), _bd(w[1, 1][IN_C:]),
            jnp.tile(b, 8))


def kernel(X, edge_index, edge_weight, Wz, bz, Wr, br, Wh, bh):
    Bs, Ts, Ns, Fin = X.shape
    row = edge_index[0].astype(jnp.int32)
    col = edge_index[1].astype(jnp.int32)
    w = edge_weight

    perm = jnp.argsort(col * Ns + row)
    src_r = col[perm]
    dst_r = row[perm]
    pad1 = E1P - E
    padi = jnp.full((pad1,), N, jnp.int32)
    padd = jnp.full((pad1,), DUMMY, jnp.int32)

    rowp = jnp.concatenate([row, padi]).reshape(E1P // SUB, SUB)
    colp = jnp.concatenate([col, padi]).reshape(E1P // SUB, SUB)
    wp = jnp.concatenate([w, jnp.zeros((pad1,), jnp.float32)]
                         ).reshape(E1P // SUB, SUB)

    def batchpad(a):
        return jnp.concatenate(
            [a, padd, a + Ns, padd]).reshape(E2P // SUB, SUB)

    srcf, dstf = batchpad(row), batchpad(col)
    srcr, dstr = batchpad(src_r), batchpad(dst_r)

    x_tab = X.transpose(0, 2, 1, 3).reshape(NN, Ts * Fin)
    x_tab = jnp.pad(x_tab, ((0, NR - NN), (0, F - Ts * Fin)))
    x_tab8 = x_tab.reshape(NR8, 128)

    packs = [(_pack_w(Wz, bz, t), _pack_w(Wr, br, t), _pack_w(Wh, bh, t))
             for t in range(Ts)]

    zeros_h = jnp.zeros((NR, F), jnp.float32)

    dinv, wr_rows = _sc_preamble(rowp, colp, wp, zeros_h)

    dinvb8, xs_x8 = _tc_scale(x_tab8, dinv.reshape(2, ND8, 128))
    pox, pix = _sc_prop(xs_x8.reshape(NR, F), x_tab, srcf, dstf, srcr, dstr,
                        wr_rows, zeros_h)
    pox8 = pox.reshape(2, NR8, 128)
    pix8 = pix.reshape(2, NR8, 128)

    h8 = jnp.zeros((NR8, 128), jnp.float32)
    h_s8 = h8
    zero_part = jnp.zeros((2, NR8, 128), jnp.float32)
    outs = []
    for t in range(Ts):
        wz_pack, wr_pack, wh_pack = packs[t]
        if t == 0:
            yo_h = yi_h = zero_part
        else:
            yo_h, yi_h = _sc_prop(h_s8.reshape(NR, F), h8.reshape(NR, F),
                                  srcf, dstf, srcr, dstr, wr_rows, zeros_h)
            yo_h = yo_h.reshape(2, NR8, 128)
            yi_h = yi_h.reshape(2, NR8, 128)
        z8, hr8, hr_s8 = _tc_gates(x_tab8, pox8, pix8, h8, yo_h, yi_h,
                                   dinvb8, wz_pack, wr_pack)
        yo_c, yi_c = _sc_prop(hr_s8.reshape(NR, F), hr8.reshape(NR, F),
                              srcf, dstf, srcr, dstr, wr_rows, zeros_h)
        h8, h_s8 = _tc_update(x_tab8, pox8, pix8, h8, z8, hr8,
                              yo_c.reshape(2, NR8, 128),
                              yi_c.reshape(2, NR8, 128), dinvb8, wh_pack)
        outs.append(h8.reshape(NR, F)[:NN].reshape(Bs, Ns, OUT_C))
    return jnp.stack(outs, axis=1)
